# Initial kernel scaffold; baseline (speedup 1.0000x reference)
import jax
import jax.numpy as jnp
from jax.experimental import pallas as pl


def kernel(x_i, x_j, recipients, W, b):
    raise NotImplementedError("write your pallas kernel here")


# speedup vs baseline: 2.4743x; 2.4743x over previous
import functools

import jax
import jax.numpy as jnp
from jax import lax
from jax.experimental import pallas as pl
from jax.experimental.pallas import tpu as pltpu
from jax.experimental.pallas import tpu_sc as plsc

_N = 10000
_NC = 2
_NS = 16
_CH = 80


def _mlp_body(xi_ref, xj_ref, w1_ref, w2_ref, b_ref, m_ref):
    acc = jnp.dot(xi_ref[...], w1_ref[...], preferred_element_type=jnp.float32)
    acc = acc + jnp.dot(xj_ref[...], w2_ref[...], preferred_element_type=jnp.float32)
    m_ref[...] = jnp.maximum(acc + b_ref[...], 0.0)


def _scatter_body(m_hbm, rec_hbm, out_hbm, idx_v, rows_v, accum):
    c = lax.axis_index("c")
    s = lax.axis_index("s")
    wid = c * _NS + s
    d = rows_v.shape[1]

    def _zrow(t, carry):
        rows_v[t // (d // 16), pl.ds((t % (d // 16)) * 16, 16)] = jnp.zeros(
            (16,), jnp.float32)
        return carry

    lax.fori_loop(0, _CH * (d // 16), _zrow, 0)

    nchunks = _N // 16

    def _zacc(i, carry):
        pltpu.sync_copy(rows_v.at[pl.ds(0, 16)],
                        accum.at[pl.ds((i * _NS + s) * 16, 16)])
        return carry

    lax.fori_loop(0, nchunks // _NS, _zacc, 0)

    @pl.when(s == 0)
    def _():
        pltpu.sync_copy(rows_v.at[pl.ds(0, 16)],
                        accum.at[pl.ds((nchunks - 1) * 16, 16)])

    plsc.subcore_barrier()

    epw = m_hbm.shape[0] // (_NC * _NS)
    ebase = wid * epw

    def _chunk(i, carry):
        o = ebase + i * _CH
        pltpu.sync_copy(rec_hbm.at[pl.ds(o, _CH)], idx_v)
        pltpu.sync_copy(m_hbm.at[pl.ds(o, _CH)], rows_v)
        pltpu.sync_copy(rows_v, accum.at[idx_v], add=True)
        return carry

    lax.fori_loop(0, epw // _CH, _chunk, 0)
    plsc.subcore_barrier()

    def _wout(i, carry):
        o = (i * _NS + s) * 16
        pltpu.sync_copy(accum.at[pl.ds(o, 16)],
                        out_hbm.at[c, pl.ds(o, 16)])
        return carry

    lax.fori_loop(0, nchunks // _NS, _wout, 0)

    @pl.when(s == 0)
    def _():
        o = (nchunks - 1) * 16
        pltpu.sync_copy(accum.at[pl.ds(o, 16)],
                        out_hbm.at[c, pl.ds(o, 16)])


def _combine_body(p_ref, o_ref):
    o_ref[...] = p_ref[0] + p_ref[1]


def kernel(x_i, x_j, recipients, W, b):
    e, d = x_i.shape
    w1 = W[:d]
    w2 = W[d:]
    b2 = b.reshape(1, d)
    rec = recipients.astype(jnp.int32)

    bm = 2560
    m = pl.pallas_call(
        _mlp_body,
        grid=(e // bm,),
        in_specs=[
            pl.BlockSpec((bm, d), lambda i: (i, 0)),
            pl.BlockSpec((bm, d), lambda i: (i, 0)),
            pl.BlockSpec((d, d), lambda i: (0, 0)),
            pl.BlockSpec((d, d), lambda i: (0, 0)),
            pl.BlockSpec((1, d), lambda i: (0, 0)),
        ],
        out_specs=pl.BlockSpec((bm, d), lambda i: (i, 0)),
        out_shape=jax.ShapeDtypeStruct((e, d), jnp.float32),
    )(x_i, x_j, w1, w2, b2)

    mesh = plsc.VectorSubcoreMesh(core_axis_name="c", subcore_axis_name="s")
    scatter = functools.partial(
        pl.kernel,
        out_type=jax.ShapeDtypeStruct((_NC, _N, d), jnp.float32),
        mesh=mesh,
        scratch_types=[
            pltpu.VMEM((_CH,), jnp.int32),
            pltpu.VMEM((_CH, d), jnp.float32),
            pltpu.VMEM_SHARED((_N, d), jnp.float32),
        ],
    )(_scatter_body)
    partials = scatter(m, rec)

    aggr = pl.pallas_call(
        _combine_body,
        out_shape=jax.ShapeDtypeStruct((_N, d), jnp.float32),
    )(partials)

    return (aggr, m)


# speedup vs baseline: 3.1695x; 1.2810x over previous
import functools

import jax
import jax.numpy as jnp
from jax import lax
from jax.experimental import pallas as pl
from jax.experimental.pallas import tpu as pltpu
from jax.experimental.pallas import tpu_sc as plsc

_N = 10000
_NC = 2
_NS = 16
_CH = 80


def _mlp_body(xi_ref, xj_ref, w1_ref, w2_ref, b_ref, m_ref):
    acc = jnp.dot(xi_ref[...], w1_ref[...], preferred_element_type=jnp.float32)
    acc = acc + jnp.dot(xj_ref[...], w2_ref[...], preferred_element_type=jnp.float32)
    m_ref[...] = jnp.maximum(acc + b_ref[...], 0.0)


def _scatter_body(m_hbm, rec_hbm, out_hbm, idx_a, rows_a, idx_b, rows_b,
                  zbuf, accum, sem_a, sem_b):
    c = lax.axis_index("c")
    s = lax.axis_index("s")
    wid = c * _NS + s
    d = rows_a.shape[1]
    epw = m_hbm.shape[0] // (_NC * _NS)
    ebase = wid * epw
    nch = epw // _CH

    def _start(j, idx_v, rows_v, sem):
        o = ebase + j * _CH
        pltpu.make_async_copy(rec_hbm.at[pl.ds(o, _CH)], idx_v, sem).start()
        pltpu.make_async_copy(m_hbm.at[pl.ds(o, _CH)], rows_v, sem).start()

    def _wait(idx_v, rows_v, sem):
        pltpu.make_async_copy(rec_hbm.at[pl.ds(0, _CH)], idx_v, sem).wait()
        pltpu.make_async_copy(m_hbm.at[pl.ds(0, _CH)], rows_v, sem).wait()

    _start(0, idx_a, rows_a, sem_a)

    def _zrow(t, carry):
        zbuf[t // (d // 16), pl.ds((t % (d // 16)) * 16, 16)] = jnp.zeros(
            (16,), jnp.float32)
        return carry

    lax.fori_loop(0, 16 * (d // 16), _zrow, 0)

    nzc = _N // 16

    def _zacc(i, carry):
        pltpu.sync_copy(zbuf, accum.at[pl.ds((i * _NS + s) * 16, 16)])
        return carry

    lax.fori_loop(0, nzc // _NS, _zacc, 0)

    @pl.when(s == 0)
    def _():
        pltpu.sync_copy(zbuf, accum.at[pl.ds((nzc - 1) * 16, 16)])

    plsc.subcore_barrier()

    def _pair(i, carry):
        j = 2 * i
        _wait(idx_a, rows_a, sem_a)
        _start(j + 1, idx_b, rows_b, sem_b)
        pltpu.sync_copy(rows_a, accum.at[idx_a], add=True)
        _wait(idx_b, rows_b, sem_b)
        _start(j + 2, idx_a, rows_a, sem_a)
        pltpu.sync_copy(rows_b, accum.at[idx_b], add=True)
        return carry

    lax.fori_loop(0, nch // 2, _pair, 0)
    _wait(idx_a, rows_a, sem_a)
    pltpu.sync_copy(rows_a, accum.at[idx_a], add=True)
    plsc.subcore_barrier()

    def _wout(i, carry):
        o = (i * _NS + s) * 16
        pltpu.sync_copy(accum.at[pl.ds(o, 16)],
                        out_hbm.at[c, pl.ds(o, 16)])
        return carry

    lax.fori_loop(0, nzc // _NS, _wout, 0)

    @pl.when(s == 0)
    def _():
        o = (nzc - 1) * 16
        pltpu.sync_copy(accum.at[pl.ds(o, 16)],
                        out_hbm.at[c, pl.ds(o, 16)])


def _combine_body(p_ref, o_ref):
    o_ref[...] = p_ref[0] + p_ref[1]


def kernel(x_i, x_j, recipients, W, b):
    e, d = x_i.shape
    w1 = W[:d]
    w2 = W[d:]
    b2 = b.reshape(1, d)
    rec = recipients.astype(jnp.int32)

    bm = 2560
    m = pl.pallas_call(
        _mlp_body,
        grid=(e // bm,),
        in_specs=[
            pl.BlockSpec((bm, d), lambda i: (i, 0)),
            pl.BlockSpec((bm, d), lambda i: (i, 0)),
            pl.BlockSpec((d, d), lambda i: (0, 0)),
            pl.BlockSpec((d, d), lambda i: (0, 0)),
            pl.BlockSpec((1, d), lambda i: (0, 0)),
        ],
        out_specs=pl.BlockSpec((bm, d), lambda i: (i, 0)),
        out_shape=jax.ShapeDtypeStruct((e, d), jnp.float32),
    )(x_i, x_j, w1, w2, b2)

    mesh = plsc.VectorSubcoreMesh(core_axis_name="c", subcore_axis_name="s")
    scatter = functools.partial(
        pl.kernel,
        out_type=jax.ShapeDtypeStruct((_NC, _N, d), jnp.float32),
        mesh=mesh,
        scratch_types=[
            pltpu.VMEM((_CH,), jnp.int32),
            pltpu.VMEM((_CH, d), jnp.float32),
            pltpu.VMEM((_CH,), jnp.int32),
            pltpu.VMEM((_CH, d), jnp.float32),
            pltpu.VMEM((16, d), jnp.float32),
            pltpu.VMEM_SHARED((_N, d), jnp.float32),
            pltpu.SemaphoreType.DMA,
            pltpu.SemaphoreType.DMA,
        ],
    )(_scatter_body)
    partials = scatter(m, rec)

    aggr = pl.pallas_call(
        _combine_body,
        out_shape=jax.ShapeDtypeStruct((_N, d), jnp.float32),
    )(partials)

    return (aggr, m)
